# Initial kernel scaffold; baseline (speedup 1.0000x reference)
import jax
import jax.numpy as jnp
from jax.experimental import pallas as pl


def kernel(H, edge_index, edge_weight, adj_s, C, M, deg, w0, w1, w2, Wa1, ba1, Wa2, ba2):
    raise NotImplementedError("write your pallas kernel here")


# speedup vs baseline: 3.7009x; 3.7009x over previous
import functools
import math

import jax
import jax.numpy as jnp
from jax import lax
from jax.experimental import pallas as pl
from jax.experimental.pallas import tpu as pltpu
from jax.experimental.pallas import tpu_sc as plsc

D = 128
RB = 1024
NCP = 10240
NW = 32
CHB = 128


def _pre_body(h_ref, c_ref, adj_ref, m_ref, htail_ref, w0_ref, w1_ref, w2_ref,
              z0_ref, hw1_ref, z2_ref):
    hb = h_ref[...]
    z0_ref[...] = jnp.maximum(
        jnp.dot(hb, w0_ref[...], preferred_element_type=jnp.float32), 0.0)
    hw1_ref[...] = jnp.dot(hb, w1_ref[...], preferred_element_type=jnp.float32)
    bs = jnp.dot(c_ref[...], m_ref[...], preferred_element_type=jnp.float32)
    abu = adj_ref[...] * bs
    s = jnp.clip(jnp.sum(jnp.abs(abu), axis=1, keepdims=True), 1e-12, None)
    ab = abu / s
    hw2 = jnp.dot(htail_ref[...], w2_ref[...], preferred_element_type=jnp.float32)
    z2_ref[...] = jnp.maximum(
        jnp.dot(ab, hw2, preferred_element_type=jnp.float32), 0.0)


def _pre(h, c, adj, m, htail, w0, w1, w2):
    grid = NCP // RB
    blk = pl.BlockSpec((RB, D), lambda i: (i, 0))
    cst = pl.BlockSpec((D, D), lambda i: (0, 0))
    return pl.pallas_call(
        _pre_body,
        grid=(grid,),
        in_specs=[blk, blk, blk, cst, cst, cst, cst, cst],
        out_specs=[blk, blk, blk],
        out_shape=[jax.ShapeDtypeStruct((NCP, D), jnp.float32)] * 3,
    )(h, c, adj, m, htail, w0, w1, w2)


def _bcast_lane(vec, lane):
    dnums = lax.GatherDimensionNumbers(
        offset_dims=(), collapsed_slice_dims=(0,), start_index_map=(0,))
    idx = jnp.full((16, 1), lane, jnp.int32)
    return lax.gather(vec, idx, dnums, (1,),
                      mode=lax.GatherScatterMode.PROMISE_IN_BOUNDS)


def _make_sc(ch):
    mesh = plsc.VectorSubcoreMesh(core_axis_name="c", subcore_axis_name="s",
                                  num_cores=2, num_subcores=16)
    rpt = NCP // 16
    nzb = rpt // CHB

    @functools.partial(
        pl.kernel,
        out_type=jax.ShapeDtypeStruct((2, NCP, D), jnp.float32),
        mesh=mesh,
        scratch_types=[
            pltpu.VMEM((ch, CHB), jnp.int32),
            pltpu.VMEM((ch, CHB), jnp.int32),
            pltpu.VMEM((ch, CHB), jnp.float32),
            pltpu.VMEM((CHB, D), jnp.float32),
            pltpu.VMEM_SHARED((NCP, D), jnp.float32),
            pltpu.SemaphoreType.DMA,
        ],
    )
    def sc_kernel(hw1_hbm, col_hbm, row_hbm, wgt_hbm, out_hbm,
                  colv, rowv, wgtv, rows_v, acc, sem):
        cid = lax.axis_index("c")
        sid = lax.axis_index("s")
        wid = sid * 2 + cid
        base = sid * rpt

        def _zrow(r, carry):
            for q in range(D // 16):
                rows_v[r, pl.ds(q * 16, 16)] = jnp.zeros((16,), jnp.float32)
            return carry
        lax.fori_loop(0, CHB, _zrow, 0)
        for b in range(nzb):
            pltpu.sync_copy(rows_v, acc.at[pl.ds(base + b * CHB, CHB)])
        plsc.subcore_barrier()

        pltpu.sync_copy(col_hbm.at[wid], colv)
        pltpu.sync_copy(row_hbm.at[wid], rowv)
        pltpu.sync_copy(wgt_hbm.at[wid], wgtv)

        def _chunk(j, carry):
            pltpu.async_copy(hw1_hbm.at[colv.at[j]], rows_v, sem).wait()

            def _grp(g, c2):
                wv = wgtv[j, pl.ds(g * 16, 16)]
                for l in range(16):
                    w16 = _bcast_lane(wv, l)
                    e = g * 16 + l
                    for q in range(D // 16):
                        rows_v[e, pl.ds(q * 16, 16)] = (
                            rows_v[e, pl.ds(q * 16, 16)] * w16)
                return c2
            lax.fori_loop(0, CHB // 16, _grp, 0)
            pltpu.sync_copy(rows_v, acc.at[rowv.at[j]], add=True)
            return carry
        lax.fori_loop(0, ch, _chunk, 0)
        plsc.subcore_barrier()

        for b in range(nzb):
            pltpu.sync_copy(acc.at[pl.ds(base + b * CHB, CHB)],
                            out_hbm.at[cid, pl.ds(base + b * CHB, CHB)])

    return sc_kernel


def _post_body(z0_ref, z1p_ref, z2_ref, deg_ref, wa1_ref, ba1_ref,
               wa2_ref, ba2_ref, z_ref):
    z0 = z0_ref[...]
    z1 = jnp.maximum(z1p_ref[0] + z1p_ref[1], 0.0)
    z2 = z2_ref[...]
    hl = (jnp.dot(z0, wa1_ref[0:128], preferred_element_type=jnp.float32)
          + jnp.dot(z1, wa1_ref[128:256], preferred_element_type=jnp.float32)
          + jnp.dot(z2, wa1_ref[256:384], preferred_element_type=jnp.float32)
          + deg_ref[:, 0:1] * wa1_ref[384:385]
          + ba1_ref[...])
    h1 = jax.nn.sigmoid(hl)
    h2 = jnp.dot(h1, wa2_ref[...], preferred_element_type=jnp.float32) + ba2_ref[...]
    colidx = lax.broadcasted_iota(jnp.int32, h2.shape, 1)
    h2 = jnp.where(colidx < 3, h2, -1e30)
    alpha = jax.nn.softmax(h2, axis=1)
    z_ref[...] = (alpha[:, 0:1] * z0 + alpha[:, 1:2] * z1 + alpha[:, 2:3] * z2)


def _post(z0, z1p, z2, deg, wa1, ba1, wa2, ba2):
    grid = NCP // RB
    blk = pl.BlockSpec((RB, D), lambda i: (i, 0))
    return pl.pallas_call(
        _post_body,
        grid=(grid,),
        in_specs=[
            blk,
            pl.BlockSpec((2, RB, D), lambda i: (0, i, 0)),
            blk,
            blk,
            pl.BlockSpec((512, D), lambda i: (0, 0)),
            pl.BlockSpec((1, D), lambda i: (0, 0)),
            pl.BlockSpec((D, D), lambda i: (0, 0)),
            pl.BlockSpec((1, D), lambda i: (0, 0)),
        ],
        out_specs=blk,
        out_shape=jax.ShapeDtypeStruct((NCP, D), jnp.float32),
    )(z0, z1p, z2, deg, wa1, ba1, wa2, ba2)


def kernel(H, edge_index, edge_weight, adj_s, C, M, deg, w0, w1, w2,
           Wa1, ba1, Wa2, ba2):
    nc = H.shape[0]
    cnum = M.shape[0]
    n = nc - cnum
    e = edge_weight.shape[0]

    f32 = jnp.float32
    h_pad = jnp.zeros((NCP, D), f32).at[:nc].set(H)
    c_pad = jnp.zeros((NCP, D), f32).at[:nc, :cnum].set(C)
    adj_pad = jnp.zeros((NCP, D), f32).at[:nc, :cnum].set(adj_s)
    m_pad = jnp.zeros((D, D), f32).at[:cnum, :cnum].set(M)
    htail_pad = jnp.zeros((D, D), f32).at[:cnum].set(H[n:])
    deg_pad = jnp.zeros((NCP, D), f32).at[:nc, 0].set(deg[:, 0])
    wa1_pad = jnp.zeros((512, D), f32).at[:3 * D + 1, :3].set(Wa1)
    ba1_pad = jnp.zeros((1, D), f32).at[0, :3].set(ba1)
    wa2_pad = jnp.zeros((D, D), f32).at[:3, :3].set(Wa2)
    ba2_pad = jnp.zeros((1, D), f32).at[0, :3].set(ba2)

    z0, hw1, z2 = _pre(h_pad, c_pad, adj_pad, m_pad, htail_pad, w0, w1, w2)

    ch = math.ceil(e / (NW * CHB))
    ep = NW * ch * CHB
    row_p = jnp.zeros((ep,), jnp.int32).at[:e].set(edge_index[0]).reshape(NW, ch, CHB)
    col_p = jnp.zeros((ep,), jnp.int32).at[:e].set(edge_index[1]).reshape(NW, ch, CHB)
    wgt_p = jnp.zeros((ep,), f32).at[:e].set(edge_weight).reshape(NW, ch, CHB)

    z1p = _make_sc(ch)(hw1, col_p, row_p, wgt_p)

    z = _post(z0, z1p, z2, deg_pad, wa1_pad, ba1_pad, wa2_pad, ba2_pad)
    return z[:nc]
